# Initial kernel scaffold; baseline (speedup 1.0000x reference)
import jax
import jax.numpy as jnp
from jax.experimental import pallas as pl


def kernel(position_ids, table):
    raise NotImplementedError("write your pallas kernel here")


# speedup vs baseline: 1.9850x; 1.9850x over previous
import functools

import jax
import jax.numpy as jnp
from jax import lax
from jax.experimental import pallas as pl
from jax.experimental.pallas import tpu as pltpu
from jax.experimental.pallas import tpu_sc as plsc

_NUM_EMBED = 8192
_DIM = 1024
_BATCH = 4
_SEQ = 8192
_B = _BATCH * _SEQ

_NC = 2
_NS = 16
_NW = _NC * _NS
_BPW = _B // _NW
_CHUNK = 32
_NCHUNK = _BPW // _CHUNK


@functools.partial(
    pl.kernel,
    mesh=plsc.VectorSubcoreMesh(core_axis_name="c", subcore_axis_name="s"),
    out_type=jax.ShapeDtypeStruct((_B, _DIM), jnp.float32),
    scratch_types=[
        pltpu.VMEM((_BPW,), jnp.int32),
        pltpu.VMEM((_CHUNK, _DIM), jnp.float32),
        pltpu.SemaphoreType.DMA,
    ],
)
def _gather_rows(ids_hbm, table_hbm, out_hbm, idx_v, buf, gsem):
    wid = lax.axis_index("s") * _NC + lax.axis_index("c")
    base = wid * _BPW
    pltpu.sync_copy(ids_hbm.at[pl.ds(base, _BPW)], idx_v)

    def step(c, carry):
        off = pl.multiple_of(c * _CHUNK, _CHUNK)
        pltpu.async_copy(
            table_hbm.at[idx_v.at[pl.ds(off, _CHUNK)]], buf, gsem
        ).wait()
        pltpu.sync_copy(buf, out_hbm.at[pl.ds(base + off, _CHUNK)])
        return carry

    lax.fori_loop(0, _NCHUNK, step, 0)


def kernel(position_ids, table):
    ids_flat = position_ids.reshape(_B)
    out = _gather_rows(ids_flat, table)
    return out.reshape(_BATCH, _SEQ, _DIM)


# speedup vs baseline: 2.3658x; 1.1919x over previous
import functools

import jax
import jax.numpy as jnp
from jax import lax
from jax.experimental import pallas as pl
from jax.experimental.pallas import tpu as pltpu
from jax.experimental.pallas import tpu_sc as plsc

_NUM_EMBED = 8192
_DIM = 1024
_BATCH = 4
_SEQ = 8192
_B = _BATCH * _SEQ

_NC = 2
_NS = 16
_NW = _NC * _NS
_BPW = _B // _NW
_CHUNK = 32
_NCHUNK = _BPW // _CHUNK


@functools.partial(
    pl.kernel,
    mesh=plsc.VectorSubcoreMesh(core_axis_name="c", subcore_axis_name="s"),
    out_type=jax.ShapeDtypeStruct((_B, _DIM), jnp.float32),
    scratch_types=[
        pltpu.VMEM((_BPW,), jnp.int32),
        pltpu.VMEM((_CHUNK, _DIM), jnp.float32),
        pltpu.VMEM((_CHUNK, _DIM), jnp.float32),
        pltpu.SemaphoreType.DMA,
        pltpu.SemaphoreType.DMA,
        pltpu.SemaphoreType.DMA,
        pltpu.SemaphoreType.DMA,
    ],
)
def _gather_rows(ids_hbm, table_hbm, out_hbm, idx_v, buf0, buf1,
                 gsem0, gsem1, wsem0, wsem1):
    wid = lax.axis_index("s") * _NC + lax.axis_index("c")
    base = wid * _BPW
    pltpu.sync_copy(ids_hbm.at[pl.ds(base, _BPW)], idx_v)

    bufs = (buf0, buf1)
    gsems = (gsem0, gsem1)
    wsems = (wsem0, wsem1)

    def start_gather(c, k):
        off = pl.multiple_of(c * _CHUNK, _CHUNK)
        pltpu.async_copy(
            table_hbm.at[idx_v.at[pl.ds(off, _CHUNK)]], bufs[k], gsems[k]
        )

    def wait_gather(c, k):
        off = pl.multiple_of(c * _CHUNK, _CHUNK)
        pltpu.make_async_copy(
            table_hbm.at[idx_v.at[pl.ds(off, _CHUNK)]], bufs[k], gsems[k]
        ).wait()

    def start_write(c, k):
        off = pl.multiple_of(c * _CHUNK, _CHUNK)
        pltpu.async_copy(bufs[k], out_hbm.at[pl.ds(base + off, _CHUNK)],
                         wsems[k])

    def wait_write(c, k):
        off = pl.multiple_of(c * _CHUNK, _CHUNK)
        pltpu.make_async_copy(
            bufs[k], out_hbm.at[pl.ds(base + off, _CHUNK)], wsems[k]
        ).wait()

    start_gather(0, 0)

    def step(i, carry):
        for k in (0, 1):
            c = 2 * i + k
            other = 1 - k

            @pl.when(c >= 1)
            def _():
                wait_write(c - 1, other)

            @pl.when(c + 1 < _NCHUNK)
            def _():
                start_gather(c + 1, other)

            wait_gather(c, k)
            start_write(c, k)
        return carry

    lax.fori_loop(0, _NCHUNK // 2, step, 0)
    wait_write(_NCHUNK - 1, 1)


def kernel(position_ids, table):
    ids_flat = position_ids.reshape(_B)
    out = _gather_rows(ids_flat, table)
    return out.reshape(_BATCH, _SEQ, _DIM)
